# baseline (speedup 1.0000x reference)
import functools

import jax
import jax.numpy as jnp
from jax import lax
from jax.experimental import pallas as pl
from jax.experimental.pallas import tpu as pltpu
from jax.experimental.pallas import tpu_sc as plsc

_N_SC_ROWS = 1024
_RBLK = 128
_NCHAIN = 8


def _argmax_body(p_ref, out_ref):
    x = p_ref[...]
    bm = jnp.max(x, axis=1, keepdims=True)
    col = jax.lax.broadcasted_iota(jnp.int32, x.shape, 1)
    big = jnp.int32(2**31 - 1)
    out_ref[...] = jnp.min(jnp.where(x == bm, col, big), axis=1, keepdims=True)


def _tc_argmax(parameter, r_tc):
    _, n_cols = parameter.shape
    grid = (r_tc // _RBLK,)
    ind2d = pl.pallas_call(
        _argmax_body,
        grid=grid,
        in_specs=[pl.BlockSpec((_RBLK, n_cols), lambda i: (i, 0))],
        out_specs=pl.BlockSpec((_RBLK, 1), lambda i: (i, 0)),
        out_shape=jax.ShapeDtypeStruct((r_tc, 1), jnp.int32),
    )(parameter)
    return ind2d.reshape(r_tc)


def _make_sc_argmax(row_base, n_sc_rows, n_cols):
    info = plsc.get_sparse_core_info()
    NC, NS = info.num_cores, info.num_subcores
    NW = NC * NS
    rpw = n_sc_rows // NW
    assert rpw % 16 == 0
    K = _NCHAIN
    n_iter = n_cols // (16 * K)
    mesh = plsc.VectorSubcoreMesh(core_axis_name="c", subcore_axis_name="s")

    @functools.partial(
        pl.kernel,
        mesh=mesh,
        out_type=jax.ShapeDtypeStruct((n_sc_rows,), jnp.int32),
        scratch_types=[
            pltpu.VMEM((n_cols,), jnp.float32),
            pltpu.VMEM((n_cols,), jnp.float32),
            pltpu.VMEM((rpw,), jnp.int32),
            pltpu.SemaphoreType.DMA,
            pltpu.SemaphoreType.DMA,
        ],
    )
    def sc_argmax(p_hbm, ind_hbm, buf0, buf1, idx_v, sem0, sem1):
        w = lax.axis_index("s") * NC + lax.axis_index("c")
        my_base = row_base + w * rpw
        bufs = (buf0, buf1)
        sems = (sem0, sem1)
        lane = lax.iota(jnp.int32, 16)
        ninf = jnp.full((16,), -jnp.inf, jnp.float32)
        zero = jnp.zeros((16,), jnp.int32)
        copies = [None, None]
        copies[0] = pltpu.async_copy(p_hbm.at[my_base], buf0, sem0)
        acc = zero

        for r in range(rpw):
            cur = r % 2
            copies[cur].wait()
            if r + 1 < rpw:
                nxt = (r + 1) % 2
                copies[nxt] = pltpu.async_copy(
                    p_hbm.at[my_base + r + 1], bufs[nxt], sems[nxt]
                )
            buf = bufs[cur]

            def chunk(j, carry, buf=buf):
                vmaxs, vchunks = carry
                nm, nc = [], []
                for k in range(K):
                    cid = j * K + k
                    v = buf[pl.ds(cid * 16, 16)]
                    m = v > vmaxs[k]
                    nm.append(jnp.where(m, v, vmaxs[k]))
                    nc.append(jnp.where(m, jnp.full((16,), cid, jnp.int32), vchunks[k]))
                return (tuple(nm), tuple(nc))

            vmaxs, vchunks = lax.fori_loop(
                0, n_iter, chunk, ((ninf,) * K, (zero,) * K)
            )
            vmax, vchunk = vmaxs[0], vchunks[0]
            for k in range(1, K):
                m = (vmaxs[k] > vmax) | ((vmaxs[k] == vmax) & (vchunks[k] < vchunk))
                vmax = jnp.where(m, vmaxs[k], vmax)
                vchunk = jnp.where(m, vchunks[k], vchunk)
            colv = vchunk * 16 + lane
            gv = vmax[0]
            gc = colv[0]
            for k in range(1, 16):
                vk = vmax[k]
                ck = colv[k]
                better = (vk > gv) | ((vk == gv) & (ck < gc))
                gv = jnp.where(better, vk, gv)
                gc = jnp.where(better, ck, gc)
            acc = jnp.where(lane == (r % 16), jnp.full((16,), gc, jnp.int32), acc)
            if r % 16 == 15:
                idx_v[pl.ds((r // 16) * 16, 16)] = acc

        pltpu.sync_copy(idx_v, ind_hbm.at[pl.ds(w * rpw, rpw)])

    return sc_argmax


def _make_sc_gather(V, D, B):
    info = plsc.get_sparse_core_info()
    NC, NS = info.num_cores, info.num_subcores
    NW = NC * NS
    assert B % (8 * NW) == 0
    b_per_w = B // NW
    mesh = plsc.VectorSubcoreMesh(core_axis_name="c", subcore_axis_name="s")

    @functools.partial(
        pl.kernel,
        mesh=mesh,
        out_type=jax.ShapeDtypeStruct((B, D), jnp.float32),
        scratch_types=[
            pltpu.VMEM((b_per_w,), jnp.int32),
            pltpu.VMEM((b_per_w, D), jnp.float32),
            pltpu.SemaphoreType.DMA,
        ],
    )
    def gather_k(table_hbm, idx_hbm, out_hbm, idx_v, rows_v, sem):
        wid = lax.axis_index("s") * NC + lax.axis_index("c")
        base = wid * b_per_w
        pltpu.sync_copy(idx_hbm.at[pl.ds(base, b_per_w)], idx_v)
        pltpu.async_copy(table_hbm.at[idx_v], rows_v, sem).wait()
        pltpu.sync_copy(rows_v, out_hbm.at[pl.ds(base, b_per_w)])

    return gather_k


def kernel(inputs, parameter):
    n_rows, n_cols = parameter.shape
    r_tc = n_rows - _N_SC_ROWS
    ind_tc = _tc_argmax(parameter, r_tc)
    ind_sc = _make_sc_argmax(r_tc, _N_SC_ROWS, n_cols)(parameter)
    ind = jnp.concatenate([ind_tc, ind_sc])
    table = inputs.T
    V, D = table.shape
    rows = _make_sc_gather(V, D, n_rows)(table, ind)
    return rows.T


# speedup vs baseline: 1.0350x; 1.0350x over previous
import functools

import jax
import jax.numpy as jnp
from jax import lax
from jax.experimental import pallas as pl
from jax.experimental.pallas import tpu as pltpu
from jax.experimental.pallas import tpu_sc as plsc

_RBLK = 128


def _argmax_body(p_ref, out_ref):
    x = p_ref[...]
    bm = jnp.max(x, axis=1, keepdims=True)
    col = jax.lax.broadcasted_iota(jnp.int32, x.shape, 1)
    big = jnp.int32(2**31 - 1)
    out_ref[...] = jnp.min(jnp.where(x == bm, col, big), axis=1)


def _rowwise_argmax(parameter):
    n_rows, n_cols = parameter.shape
    grid = (n_rows // _RBLK,)
    return pl.pallas_call(
        _argmax_body,
        grid=grid,
        in_specs=[pl.BlockSpec((_RBLK, n_cols), lambda i: (i, 0))],
        out_specs=pl.BlockSpec((_RBLK,), lambda i: (i,)),
        out_shape=jax.ShapeDtypeStruct((n_rows,), jnp.int32),
    )(parameter)


def _make_sc_gather(V, D, B):
    info = plsc.get_sparse_core_info()
    NC, NS = info.num_cores, info.num_subcores
    NW = NC * NS
    assert B % (8 * NW) == 0
    b_per_w = B // NW
    mesh = plsc.VectorSubcoreMesh(core_axis_name="c", subcore_axis_name="s")

    @functools.partial(
        pl.kernel,
        mesh=mesh,
        out_type=jax.ShapeDtypeStruct((B, D), jnp.float32),
        scratch_types=[
            pltpu.VMEM((b_per_w,), jnp.int32),
            pltpu.VMEM((b_per_w, D), jnp.float32),
            pltpu.SemaphoreType.DMA,
        ],
    )
    def gather_k(table_hbm, idx_hbm, out_hbm, idx_v, rows_v, sem):
        wid = lax.axis_index("s") * NC + lax.axis_index("c")
        base = wid * b_per_w
        pltpu.sync_copy(idx_hbm.at[pl.ds(base, b_per_w)], idx_v)
        pltpu.async_copy(table_hbm.at[idx_v], rows_v, sem).wait()
        pltpu.sync_copy(rows_v, out_hbm.at[pl.ds(base, b_per_w)])

    return gather_k


def kernel(inputs, parameter):
    ind = _rowwise_argmax(parameter)
    table = inputs.T
    V, D = table.shape
    B = ind.shape[0]
    rows = _make_sc_gather(V, D, B)(table, ind)
    return rows.T


# speedup vs baseline: 1.0450x; 1.0096x over previous
import functools

import jax
import jax.numpy as jnp
from jax import lax
from jax.experimental import pallas as pl
from jax.experimental.pallas import tpu as pltpu
from jax.experimental.pallas import tpu_sc as plsc

_RBLK = 128


def _argmax_body(p_ref, in_ref, out_ref, tab_ref):
    x = p_ref[...]
    bm = jnp.max(x, axis=1, keepdims=True)
    col = jax.lax.broadcasted_iota(jnp.int32, x.shape, 1)
    big = jnp.int32(2**31 - 1)
    out_ref[...] = jnp.min(jnp.where(x == bm, col, big), axis=1)
    tab_ref[...] = in_ref[...].T


def _rowwise_argmax(parameter, inputs):
    n_rows, n_cols = parameter.shape
    n_b, n_in = inputs.shape
    grid = (n_rows // _RBLK,)
    tcol = n_in // grid[0]
    return pl.pallas_call(
        _argmax_body,
        grid=grid,
        in_specs=[
            pl.BlockSpec((_RBLK, n_cols), lambda i: (i, 0)),
            pl.BlockSpec((n_b, tcol), lambda i: (0, i)),
        ],
        out_specs=[
            pl.BlockSpec((_RBLK,), lambda i: (i,)),
            pl.BlockSpec((tcol, n_b), lambda i: (i, 0)),
        ],
        out_shape=[
            jax.ShapeDtypeStruct((n_rows,), jnp.int32),
            jax.ShapeDtypeStruct((n_in, n_b), jnp.float32),
        ],
    )(parameter, inputs)


def _make_sc_gather(V, D, B):
    info = plsc.get_sparse_core_info()
    NC, NS = info.num_cores, info.num_subcores
    NW = NC * NS
    assert B % (8 * NW) == 0
    b_per_w = B // NW
    mesh = plsc.VectorSubcoreMesh(core_axis_name="c", subcore_axis_name="s")

    @functools.partial(
        pl.kernel,
        mesh=mesh,
        out_type=jax.ShapeDtypeStruct((B, D), jnp.float32),
        scratch_types=[
            pltpu.VMEM((b_per_w,), jnp.int32),
            pltpu.VMEM((b_per_w, D), jnp.float32),
            pltpu.SemaphoreType.DMA,
        ],
    )
    def gather_k(table_hbm, idx_hbm, out_hbm, idx_v, rows_v, sem):
        wid = lax.axis_index("s") * NC + lax.axis_index("c")
        base = wid * b_per_w
        pltpu.sync_copy(idx_hbm.at[pl.ds(base, b_per_w)], idx_v)
        pltpu.async_copy(table_hbm.at[idx_v], rows_v, sem).wait()
        pltpu.sync_copy(rows_v, out_hbm.at[pl.ds(base, b_per_w)])

    return gather_k


def kernel(inputs, parameter):
    ind, table = _rowwise_argmax(parameter, inputs)
    V, D = table.shape
    B = ind.shape[0]
    rows = _make_sc_gather(V, D, B)(table, ind)
    return rows.T


# speedup vs baseline: 1.0455x; 1.0005x over previous
import functools

import jax
import jax.numpy as jnp
from jax import lax
from jax.experimental import pallas as pl
from jax.experimental.pallas import tpu as pltpu
from jax.experimental.pallas import tpu_sc as plsc

_RBLK = 128


def _argmax_body(pa_ref, pb_ref, in_ref, out_ref, tab_ref):
    big = jnp.int32(2**31 - 1)

    def half_argmax(x, off):
        bm = jnp.max(x, axis=1, keepdims=True)
        col = jax.lax.broadcasted_iota(jnp.int32, x.shape, 1) + off
        return bm[:, 0], jnp.min(jnp.where(x == bm, col, big), axis=1)

    h = pa_ref.shape[1]
    ma, ia = half_argmax(pa_ref[...], 0)
    mb, ib = half_argmax(pb_ref[...], h)
    out_ref[...] = jnp.where(mb > ma, ib, ia)
    tab_ref[...] = in_ref[...].T


def _rowwise_argmax(parameter, inputs):
    n_rows, n_cols = parameter.shape
    n_b, n_in = inputs.shape
    grid = (n_rows // _RBLK,)
    tcol = n_in // grid[0]
    h = n_cols // 2
    return pl.pallas_call(
        _argmax_body,
        grid=grid,
        in_specs=[
            pl.BlockSpec((_RBLK, h), lambda i: (i, 0)),
            pl.BlockSpec((_RBLK, h), lambda i: (i, 1)),
            pl.BlockSpec((n_b, tcol), lambda i: (0, i)),
        ],
        out_specs=[
            pl.BlockSpec((_RBLK,), lambda i: (i,)),
            pl.BlockSpec((tcol, n_b), lambda i: (i, 0)),
        ],
        out_shape=[
            jax.ShapeDtypeStruct((n_rows,), jnp.int32),
            jax.ShapeDtypeStruct((n_in, n_b), jnp.float32),
        ],
    )(parameter, parameter, inputs)


def _make_sc_gather(V, D, B):
    info = plsc.get_sparse_core_info()
    NC, NS = info.num_cores, info.num_subcores
    NW = NC * NS
    assert B % (8 * NW) == 0
    b_per_w = B // NW
    mesh = plsc.VectorSubcoreMesh(core_axis_name="c", subcore_axis_name="s")

    @functools.partial(
        pl.kernel,
        mesh=mesh,
        out_type=jax.ShapeDtypeStruct((B, D), jnp.float32),
        scratch_types=[
            pltpu.VMEM((b_per_w,), jnp.int32),
            pltpu.VMEM((b_per_w, D), jnp.float32),
            pltpu.SemaphoreType.DMA,
        ],
    )
    def gather_k(table_hbm, idx_hbm, out_hbm, idx_v, rows_v, sem):
        wid = lax.axis_index("s") * NC + lax.axis_index("c")
        base = wid * b_per_w
        pltpu.sync_copy(idx_hbm.at[pl.ds(base, b_per_w)], idx_v)
        pltpu.async_copy(table_hbm.at[idx_v], rows_v, sem).wait()
        pltpu.sync_copy(rows_v, out_hbm.at[pl.ds(base, b_per_w)])

    return gather_k


def kernel(inputs, parameter):
    ind, table = _rowwise_argmax(parameter, inputs)
    V, D = table.shape
    B = ind.shape[0]
    rows = _make_sc_gather(V, D, B)(table, ind)
    return rows.T
